# baseline (speedup 1.0000x reference)
import functools

import jax
import jax.numpy as jnp
from jax import lax
from jax.experimental import pallas as pl
from jax.experimental.pallas import tpu as pltpu
from jax.experimental.pallas import tpu_sc as plsc

N, C, H, W = 256, 128, 32, 32
HW = H * W
K = 8
EPS = 1e-6
ALPHA = 0.1
LANES = 16
CG = C // LANES
BN = 16


def _stats_body(x_ref, xs_ref, xsq_ref):
    xb = x_ref[...]
    xs_ref[...] = jnp.sum(xb, axis=(1, 2))
    xsq_ref[...] = jnp.sum(xb * xb, axis=(1, 2))


SC_N0 = 128

_stats_call = pl.pallas_call(
    _stats_body,
    grid=(SC_N0 // BN,),
    in_specs=[pl.BlockSpec((BN, H, W, C), lambda i: (i, 0, 0, 0))],
    out_specs=[pl.BlockSpec((BN, C), lambda i: (i, 0)),
               pl.BlockSpec((BN, C), lambda i: (i, 0))],
    out_shape=[jax.ShapeDtypeStruct((SC_N0, C), jnp.float32),
               jax.ShapeDtypeStruct((SC_N0, C), jnp.float32)],
    compiler_params=pltpu.CompilerParams(dimension_semantics=("parallel",)),
)


PER_W = (N - SC_N0) // 32
CH = 8
NCH = H // CH


def _sc_stats_body(x_hbm, xs_hbm, xsq_hbm, buf0, buf1, out_xs, out_xsq,
                   sem0, sem1):
    cid = lax.axis_index("c")
    sid = lax.axis_index("s")
    wid = sid * 2 + cid
    base = SC_N0 + wid * PER_W
    bufs = (buf0, buf1)
    sems = (sem0, sem1)

    def chunk_copy(n, c, b):
        return pltpu.make_async_copy(
            x_hbm.at[n, pl.ds(c * CH, CH)], bufs[b], sems[b])

    chunk_copy(base, 0, 0).start()

    def sample_body(j, carry):
        n = base + j
        zero = jnp.zeros((LANES,), jnp.float32)
        accs = [zero] * (2 * CG)
        for c in range(NCH):
            b = c % 2
            nb = (c + 1) % 2
            if c + 1 < NCH:
                chunk_copy(n, c + 1, nb).start()
            else:
                @pl.when(j + 1 < PER_W)
                def _():
                    chunk_copy(n + 1, 0, nb).start()
            chunk_copy(n, c, b).wait()
            buf = bufs[b]

            def h_body(h, hcarry):
                a = list(hcarry)
                for w in range(W):
                    for cg in range(CG):
                        v = buf[h, w, pl.ds(cg * LANES, LANES)]
                        a[cg] = a[cg] + v
                        a[CG + cg] = a[CG + cg] + v * v
                return tuple(a)

            accs = list(lax.fori_loop(0, CH, h_body, tuple(accs)))
        for cg in range(CG):
            out_xs[j, pl.ds(cg * LANES, LANES)] = accs[cg]
            out_xsq[j, pl.ds(cg * LANES, LANES)] = accs[CG + cg]
        return carry

    lax.fori_loop(0, PER_W, sample_body, 0)

    pltpu.sync_copy(out_xs, xs_hbm.at[pl.ds(wid * PER_W, PER_W), :])
    pltpu.sync_copy(out_xsq, xsq_hbm.at[pl.ds(wid * PER_W, PER_W), :])


@functools.lru_cache(maxsize=1)
def _sc_stats_call():
    return pl.kernel(
        _sc_stats_body,
        out_type=(jax.ShapeDtypeStruct((N - SC_N0, C), jnp.float32),
                  jax.ShapeDtypeStruct((N - SC_N0, C), jnp.float32)),
        mesh=plsc.VectorSubcoreMesh(core_axis_name="c", subcore_axis_name="s"),
        scratch_types=[
            pltpu.VMEM((CH, W, C), jnp.float32),
            pltpu.VMEM((CH, W, C), jnp.float32),
            pltpu.VMEM((PER_W, C), jnp.float32),
            pltpu.VMEM((PER_W, C), jnp.float32),
            pltpu.SemaphoreType.DMA,
            pltpu.SemaphoreType.DMA,
        ],
        compiler_params=pltpu.CompilerParams(use_tc_tiling_on_sc=False),
    )


def _sc_body(cmT_hbm, xs1_hbm, xsq1_hbm, xs2_hbm, xsq2_hbm,
             cmu_hbm, cvar_hbm, xsall_hbm, xsqall_hbm,
             cm_v, labels_v, counts_s,
             xs_v, xsq_v, ssum_v, ssq_v, cmu_v, cvar_v, omu_v, ovar_v):
    cid = lax.axis_index("c")
    sid = lax.axis_index("s")
    wid = sid * 2 + cid

    @pl.when(wid < CG)
    def _():
        g = wid * LANES
        pltpu.sync_copy(cmT_hbm, cm_v)
        pltpu.sync_copy(xs1_hbm.at[:, pl.ds(g, LANES)],
                        xs_v.at[pl.ds(0, SC_N0), :])
        pltpu.sync_copy(xsq1_hbm.at[:, pl.ds(g, LANES)],
                        xsq_v.at[pl.ds(0, SC_N0), :])
        pltpu.sync_copy(xs2_hbm.at[:, pl.ds(g, LANES)],
                        xs_v.at[pl.ds(SC_N0, N - SC_N0), :])
        pltpu.sync_copy(xsq2_hbm.at[:, pl.ds(g, LANES)],
                        xsq_v.at[pl.ds(SC_N0, N - SC_N0), :])

        for i in range(N // LANES):
            b = i * LANES
            best = cm_v[0, pl.ds(b, LANES)]
            besti = jnp.zeros((LANES,), jnp.int32)
            for k in range(1, K):
                v = cm_v[k, pl.ds(b, LANES)]
                m = v > best
                best = jnp.where(m, v, best)
                besti = jnp.where(m, k, besti)
            labels_v[pl.ds(b, LANES)] = besti

        zeros = jnp.zeros((LANES,), jnp.float32)
        for k in range(K):
            ssum_v[k, :] = zeros
            ssq_v[k, :] = zeros
            counts_s[k] = 0

        def seg_body(n, carry):
            l = labels_v[pl.ds(n, LANES)][0]
            counts_s[l] = counts_s[l] + 1
            ssum_v[l, :] = ssum_v[l, :] + xs_v[n, :]
            ssq_v[l, :] = ssq_v[l, :] + xsq_v[n, :]
            return carry
        lax.fori_loop(0, N, seg_body, 0)

        for k in range(K):
            nk = (counts_s[k] * HW).astype(jnp.float32)
            mu = ssum_v[k, :] / jnp.maximum(nk, 1.0)
            var = (ssq_v[k, :] - nk * mu * mu) / jnp.maximum(nk - 1.0, 1.0)
            cmu_v[k, :] = mu
            cvar_v[k, :] = var

        def gat_body(n, carry):
            l = labels_v[pl.ds(n, LANES)][0]
            omu_v[n, :] = cmu_v[l, :]
            ovar_v[n, :] = cvar_v[l, :]
            return carry
        lax.fori_loop(0, N, gat_body, 0)

        pltpu.sync_copy(omu_v, cmu_hbm.at[:, pl.ds(g, LANES)])
        pltpu.sync_copy(ovar_v, cvar_hbm.at[:, pl.ds(g, LANES)])
        pltpu.sync_copy(xs_v, xsall_hbm.at[:, pl.ds(g, LANES)])
        pltpu.sync_copy(xsq_v, xsqall_hbm.at[:, pl.ds(g, LANES)])


def _sc_scratch_types():
    return [
        pltpu.VMEM((K, N), jnp.float32),
        pltpu.VMEM((N + LANES,), jnp.int32),
        pltpu.SMEM((K,), jnp.int32),
        pltpu.VMEM((N, LANES), jnp.float32),
        pltpu.VMEM((N, LANES), jnp.float32),
        pltpu.VMEM((K, LANES), jnp.float32),
        pltpu.VMEM((K, LANES), jnp.float32),
        pltpu.VMEM((K, LANES), jnp.float32),
        pltpu.VMEM((K, LANES), jnp.float32),
        pltpu.VMEM((N, LANES), jnp.float32),
        pltpu.VMEM((N, LANES), jnp.float32),
    ]


@functools.lru_cache(maxsize=1)
def _sc_cluster_call():
    return pl.kernel(
        _sc_body,
        out_type=(jax.ShapeDtypeStruct((N, C), jnp.float32),
                  jax.ShapeDtypeStruct((N, C), jnp.float32),
                  jax.ShapeDtypeStruct((N, C), jnp.float32),
                  jax.ShapeDtypeStruct((N, C), jnp.float32)),
        mesh=plsc.VectorSubcoreMesh(core_axis_name="c", subcore_axis_name="s"),
        scratch_types=_sc_scratch_types(),
        compiler_params=pltpu.CompilerParams(use_tc_tiling_on_sc=False),
    )


def _apply_body(x_ref, xs_ref, xsq_ref, cmu_ref, cvar_ref, lmda_ref, out_ref):
    xs = xs_ref[...]
    smu = xs * (1.0 / HW)
    svar = (xsq_ref[...] - HW * smu * smu) * (1.0 / (HW - 1))
    sstd = jnp.sqrt(svar + EPS)
    cstd = jnp.sqrt(cvar_ref[...] + EPS)
    lm = lmda_ref[...]
    om = 1.0 - lm
    std_mix = sstd * lm + cstd * om
    mu_mix = smu * lm + cmu_ref[...] * om
    scale = std_mix / sstd
    shift = mu_mix - smu * scale
    out_ref[...] = x_ref[...] * scale[:, None, None, :] + shift[:, None, None, :]


_apply_call = pl.pallas_call(
    _apply_body,
    grid=(N // BN,),
    in_specs=[pl.BlockSpec((BN, H, W, C), lambda i: (i, 0, 0, 0)),
              pl.BlockSpec((BN, C), lambda i: (i, 0)),
              pl.BlockSpec((BN, C), lambda i: (i, 0)),
              pl.BlockSpec((BN, C), lambda i: (i, 0)),
              pl.BlockSpec((BN, C), lambda i: (i, 0)),
              pl.BlockSpec((BN, 1), lambda i: (i, 0))],
    out_specs=pl.BlockSpec((BN, H, W, C), lambda i: (i, 0, 0, 0)),
    out_shape=jax.ShapeDtypeStruct((N, H, W, C), jnp.float32),
    compiler_params=pltpu.CompilerParams(dimension_semantics=("parallel",)),
)


def kernel(x, cluster_map):
    xt = jnp.transpose(x, (0, 2, 3, 1))
    cmT = cluster_map[0].T
    xs1, xsq1 = _stats_call(xt)
    xs2, xsq2 = _sc_stats_call()(xt)
    cmu, cvar, xs, xsq = _sc_cluster_call()(cmT, xs1, xsq1, xs2, xsq2)
    lkey = jax.random.fold_in(jax.random.key(0), 12345)
    lmda = jax.random.beta(lkey, ALPHA, ALPHA, (N, 1, 1, 1)).astype(x.dtype)
    out = _apply_call(xt, xs, xsq, cmu, cvar, lmda.reshape(N, 1))
    return jnp.transpose(out, (0, 3, 1, 2))


# speedup vs baseline: 1.5773x; 1.5773x over previous
import functools

import jax
import jax.numpy as jnp
from jax import lax
from jax.experimental import pallas as pl
from jax.experimental.pallas import tpu as pltpu
from jax.experimental.pallas import tpu_sc as plsc

N, C, H, W = 256, 128, 32, 32
HW = H * W
K = 8
EPS = 1e-6
ALPHA = 0.1
LANES = 16
CG = C // LANES
BN = 16


def _stats_body(x_ref, xs_ref, xsq_ref):
    xb = x_ref[...]
    xs_ref[...] = jnp.sum(xb, axis=(1, 2))
    xsq_ref[...] = jnp.sum(xb * xb, axis=(1, 2))


SC_N0 = 128

_stats_call = pl.pallas_call(
    _stats_body,
    grid=(SC_N0 // BN,),
    in_specs=[pl.BlockSpec((BN, H, W, C), lambda i: (i, 0, 0, 0))],
    out_specs=[pl.BlockSpec((BN, C), lambda i: (i, 0)),
               pl.BlockSpec((BN, C), lambda i: (i, 0))],
    out_shape=[jax.ShapeDtypeStruct((SC_N0, C), jnp.float32),
               jax.ShapeDtypeStruct((SC_N0, C), jnp.float32)],
    compiler_params=pltpu.CompilerParams(dimension_semantics=("parallel",)),
)


PER_W = (N - SC_N0) // 32
CH = 8
NCH = H // CH


def _sc_stats_body(x_hbm, xs_hbm, xsq_hbm, buf0, buf1, out_xs, out_xsq,
                   sem0, sem1):
    cid = lax.axis_index("c")
    sid = lax.axis_index("s")
    wid = sid * 2 + cid
    base = SC_N0 + wid * PER_W
    bufs = (buf0, buf1)
    sems = (sem0, sem1)

    def chunk_copy(n, c, b):
        return pltpu.make_async_copy(
            x_hbm.at[n, pl.ds(c * CH, CH)], bufs[b], sems[b])

    chunk_copy(base, 0, 0).start()

    def sample_body(j, carry):
        n = base + j
        zero = jnp.zeros((LANES,), jnp.float32)
        accs = [zero] * (2 * CG)
        for c in range(NCH):
            b = c % 2
            nb = (c + 1) % 2
            if c + 1 < NCH:
                chunk_copy(n, c + 1, nb).start()
            else:
                @pl.when(j + 1 < PER_W)
                def _():
                    chunk_copy(n + 1, 0, nb).start()
            chunk_copy(n, c, b).wait()
            buf = bufs[b]

            def h_body(h, hcarry):
                a = list(hcarry)
                for w in range(W):
                    for cg in range(CG):
                        v = buf[h, w, pl.ds(cg * LANES, LANES)]
                        a[cg] = a[cg] + v
                        a[CG + cg] = a[CG + cg] + v * v
                return tuple(a)

            accs = list(lax.fori_loop(0, CH, h_body, tuple(accs)))
        for cg in range(CG):
            out_xs[j, pl.ds(cg * LANES, LANES)] = accs[cg]
            out_xsq[j, pl.ds(cg * LANES, LANES)] = accs[CG + cg]
        return carry

    lax.fori_loop(0, PER_W, sample_body, 0)

    pltpu.sync_copy(out_xs, xs_hbm.at[pl.ds(wid * PER_W, PER_W), :])
    pltpu.sync_copy(out_xsq, xsq_hbm.at[pl.ds(wid * PER_W, PER_W), :])


@functools.lru_cache(maxsize=1)
def _sc_stats_call():
    return pl.kernel(
        _sc_stats_body,
        out_type=(jax.ShapeDtypeStruct((N - SC_N0, C), jnp.float32),
                  jax.ShapeDtypeStruct((N - SC_N0, C), jnp.float32)),
        mesh=plsc.VectorSubcoreMesh(core_axis_name="c", subcore_axis_name="s"),
        scratch_types=[
            pltpu.VMEM((CH, W, C), jnp.float32),
            pltpu.VMEM((CH, W, C), jnp.float32),
            pltpu.VMEM((PER_W, C), jnp.float32),
            pltpu.VMEM((PER_W, C), jnp.float32),
            pltpu.SemaphoreType.DMA,
            pltpu.SemaphoreType.DMA,
        ],
        compiler_params=pltpu.CompilerParams(use_tc_tiling_on_sc=False),
    )


def _sc_body(cmT_hbm, xs1_hbm, xsq1_hbm, xs2_hbm, xsq2_hbm,
             cmu_hbm, cvar_hbm, xsall_hbm, xsqall_hbm,
             cm_v, labels_v, counts_s,
             xs_v, xsq_v, ssum_v, ssq_v, cmu_v, cvar_v, omu_v, ovar_v):
    cid = lax.axis_index("c")
    sid = lax.axis_index("s")
    wid = sid * 2 + cid

    @pl.when(wid < CG)
    def _():
        g = wid * LANES
        pltpu.sync_copy(cmT_hbm, cm_v)
        pltpu.sync_copy(xs1_hbm.at[:, pl.ds(g, LANES)],
                        xs_v.at[pl.ds(0, SC_N0), :])
        pltpu.sync_copy(xsq1_hbm.at[:, pl.ds(g, LANES)],
                        xsq_v.at[pl.ds(0, SC_N0), :])
        pltpu.sync_copy(xs2_hbm.at[:, pl.ds(g, LANES)],
                        xs_v.at[pl.ds(SC_N0, N - SC_N0), :])
        pltpu.sync_copy(xsq2_hbm.at[:, pl.ds(g, LANES)],
                        xsq_v.at[pl.ds(SC_N0, N - SC_N0), :])

        for i in range(N // LANES):
            b = i * LANES
            best = cm_v[0, pl.ds(b, LANES)]
            besti = jnp.zeros((LANES,), jnp.int32)
            for k in range(1, K):
                v = cm_v[k, pl.ds(b, LANES)]
                m = v > best
                best = jnp.where(m, v, best)
                besti = jnp.where(m, k, besti)
            labels_v[pl.ds(b, LANES)] = besti

        zeros = jnp.zeros((LANES,), jnp.float32)
        for k in range(K):
            ssum_v[k, :] = zeros
            ssq_v[k, :] = zeros
            counts_s[k] = 0

        def seg_body(n, carry):
            l = labels_v[pl.ds(n, LANES)][0]
            counts_s[l] = counts_s[l] + 1
            ssum_v[l, :] = ssum_v[l, :] + xs_v[n, :]
            ssq_v[l, :] = ssq_v[l, :] + xsq_v[n, :]
            return carry
        lax.fori_loop(0, N, seg_body, 0)

        for k in range(K):
            nk = (counts_s[k] * HW).astype(jnp.float32)
            mu = ssum_v[k, :] / jnp.maximum(nk, 1.0)
            var = (ssq_v[k, :] - nk * mu * mu) / jnp.maximum(nk - 1.0, 1.0)
            cmu_v[k, :] = mu
            cvar_v[k, :] = var

        def gat_body(n, carry):
            l = labels_v[pl.ds(n, LANES)][0]
            omu_v[n, :] = cmu_v[l, :]
            ovar_v[n, :] = cvar_v[l, :]
            return carry
        lax.fori_loop(0, N, gat_body, 0)

        pltpu.sync_copy(omu_v, cmu_hbm.at[:, pl.ds(g, LANES)])
        pltpu.sync_copy(ovar_v, cvar_hbm.at[:, pl.ds(g, LANES)])
        pltpu.sync_copy(xs_v, xsall_hbm.at[:, pl.ds(g, LANES)])
        pltpu.sync_copy(xsq_v, xsqall_hbm.at[:, pl.ds(g, LANES)])


def _sc_scratch_types():
    return [
        pltpu.VMEM((K, N), jnp.float32),
        pltpu.VMEM((N + LANES,), jnp.int32),
        pltpu.SMEM((K,), jnp.int32),
        pltpu.VMEM((N, LANES), jnp.float32),
        pltpu.VMEM((N, LANES), jnp.float32),
        pltpu.VMEM((K, LANES), jnp.float32),
        pltpu.VMEM((K, LANES), jnp.float32),
        pltpu.VMEM((K, LANES), jnp.float32),
        pltpu.VMEM((K, LANES), jnp.float32),
        pltpu.VMEM((N, LANES), jnp.float32),
        pltpu.VMEM((N, LANES), jnp.float32),
    ]


@functools.lru_cache(maxsize=1)
def _sc_cluster_call():
    return pl.kernel(
        _sc_body,
        out_type=(jax.ShapeDtypeStruct((N, C), jnp.float32),
                  jax.ShapeDtypeStruct((N, C), jnp.float32),
                  jax.ShapeDtypeStruct((N, C), jnp.float32),
                  jax.ShapeDtypeStruct((N, C), jnp.float32)),
        mesh=plsc.VectorSubcoreMesh(core_axis_name="c", subcore_axis_name="s"),
        scratch_types=_sc_scratch_types(),
        compiler_params=pltpu.CompilerParams(use_tc_tiling_on_sc=False),
    )


def _apply_body(x_ref, xs_ref, xsq_ref, cmu_ref, cvar_ref, lmda_ref, out_ref):
    xs = xs_ref[...]
    smu = xs * (1.0 / HW)
    svar = (xsq_ref[...] - HW * smu * smu) * (1.0 / (HW - 1))
    sstd = jnp.sqrt(svar + EPS)
    cstd = jnp.sqrt(cvar_ref[...] + EPS)
    lm = lmda_ref[...]
    om = 1.0 - lm
    std_mix = sstd * lm + cstd * om
    mu_mix = smu * lm + cmu_ref[...] * om
    scale = std_mix / sstd
    shift = mu_mix - smu * scale
    out_ref[...] = x_ref[...] * scale[:, None, None, :] + shift[:, None, None, :]


_apply_call = pl.pallas_call(
    _apply_body,
    grid=(N // BN,),
    in_specs=[pl.BlockSpec((BN, H, W, C), lambda i: (i, 0, 0, 0)),
              pl.BlockSpec((BN, C), lambda i: (i, 0)),
              pl.BlockSpec((BN, C), lambda i: (i, 0)),
              pl.BlockSpec((BN, C), lambda i: (i, 0)),
              pl.BlockSpec((BN, C), lambda i: (i, 0)),
              pl.BlockSpec((BN, 1), lambda i: (i, 0))],
    out_specs=pl.BlockSpec((BN, H, W, C), lambda i: (i, 0, 0, 0)),
    out_shape=jax.ShapeDtypeStruct((N, H, W, C), jnp.float32),
    compiler_params=pltpu.CompilerParams(dimension_semantics=("parallel",)),
)


@functools.lru_cache(maxsize=1)
def _lmda_const():
    import numpy as np
    with jax.ensure_compile_time_eval():
        lkey = jax.random.fold_in(jax.random.key(0), 12345)
        vals = jax.random.beta(lkey, ALPHA, ALPHA, (N, 1, 1, 1))
    return np.asarray(vals, dtype=np.float32).reshape(N, 1)


def kernel(x, cluster_map):
    xt = jnp.transpose(x, (0, 2, 3, 1))
    cmT = cluster_map[0].T
    xs1, xsq1 = _stats_call(xt)
    xs2, xsq2 = _sc_stats_call()(xt)
    cmu, cvar, xs, xsq = _sc_cluster_call()(cmT, xs1, xsq1, xs2, xsq2)
    lmda = jnp.asarray(_lmda_const(), dtype=x.dtype)
    out = _apply_call(xt, xs, xsq, cmu, cvar, lmda)
    return jnp.transpose(out, (0, 3, 1, 2))
